# Initial kernel scaffold; baseline (speedup 1.0000x reference)
import jax
import jax.numpy as jnp
from jax.experimental import pallas as pl


def kernel(hv, edge_index, W, b, gamma, beta):
    raise NotImplementedError("write your pallas kernel here")


# speedup vs baseline: 1.9679x; 1.9679x over previous
import functools

import jax
import jax.numpy as jnp
from jax import lax
from jax.experimental import pallas as pl
from jax.experimental.pallas import tpu as pltpu
from jax.experimental.pallas import tpu_sc as plsc

N = 10000
E = 160000
R = 3
D = 256
EPS = 1e-5

NC = 2
NS = 16
CH = 128
HALF = D // 2
NP_NODES = 10240
ROWS_PER_TILE = NP_NODES // NS

M = R * E
KB = 16
K = -(-(-(-M // (NS * CH))) // KB) * KB
NB = K // KB
MP = NS * K * CH


def _mm_body(hv_ref, w_ref, b_ref, out_ref):
    out_ref[0] = (
        jnp.dot(hv_ref[...], w_ref[0, 0], preferred_element_type=jnp.float32)
        + b_ref[0, 0, 0][None, :]
    )


def _make_table(hv, W2, b2):
    BN = 400
    nb = N // BN
    return pl.pallas_call(
        _mm_body,
        grid=(NC, R, nb),
        in_specs=[
            pl.BlockSpec((BN, D), lambda c, r, i: (i, 0)),
            pl.BlockSpec((1, 1, D, HALF), lambda c, r, i: (c, r, 0, 0)),
            pl.BlockSpec((1, 1, 1, HALF), lambda c, r, i: (c, r, 0, 0)),
        ],
        out_specs=pl.BlockSpec((1, BN, HALF), lambda c, r, i: (c, r * nb + i, 0)),
        out_shape=jax.ShapeDtypeStruct((NC, R * N, HALF), jnp.float32),
    )(hv, W2, b2)


def _sc_body(table_hbm, src_hbm, dst_hbm, zeros_hbm, eu_hbm,
             src_v, dst_v, rows_v, acc, sem):
    c = lax.axis_index("c")
    s = lax.axis_index("s")
    pltpu.sync_copy(zeros_hbm, acc.at[pl.ds(s * ROWS_PER_TILE, ROWS_PER_TILE)])
    plsc.subcore_barrier()

    def outer(blk, carry):
        pltpu.sync_copy(src_hbm.at[s].at[pl.ds(blk * KB, KB)], src_v)
        pltpu.sync_copy(dst_hbm.at[s].at[pl.ds(blk * KB, KB)], dst_v)

        def step(j, carry2):
            pltpu.async_copy(table_hbm.at[c].at[src_v.at[j]], rows_v, sem).wait()
            pltpu.sync_copy(rows_v, acc.at[dst_v.at[j]], add=True)
            return carry2

        return lax.fori_loop(0, KB, step, carry)

    lax.fori_loop(0, NB, outer, 0)
    plsc.subcore_barrier()
    pltpu.sync_copy(
        acc.at[pl.ds(s * ROWS_PER_TILE, ROWS_PER_TILE)],
        eu_hbm.at[c].at[pl.ds(s * ROWS_PER_TILE, ROWS_PER_TILE)],
    )


_sc_scatter = functools.partial(
    pl.kernel,
    out_type=jax.ShapeDtypeStruct((NC, NP_NODES, HALF), jnp.float32),
    mesh=plsc.VectorSubcoreMesh(core_axis_name="c", subcore_axis_name="s"),
    scratch_types=[
        pltpu.VMEM((KB, CH), jnp.int32),
        pltpu.VMEM((KB, CH), jnp.int32),
        pltpu.VMEM((CH, HALF), jnp.float32),
        pltpu.VMEM_SHARED((NP_NODES, HALF), jnp.float32),
        pltpu.SemaphoreType.DMA,
    ],
)(_sc_body)


def _stats_body(eu_ref, sum_ref, sq_ref, acc_ref):
    i = pl.program_id(0)

    @pl.when(i == 0)
    def _():
        acc_ref[...] = jnp.zeros_like(acc_ref)

    x = eu_ref[...]
    acc_ref[0] += jnp.sum(x, axis=1)
    acc_ref[1] += jnp.sum(x * x, axis=1)

    @pl.when(i == pl.num_programs(0) - 1)
    def _():
        sum_ref[...] = acc_ref[0]
        sq_ref[...] = acc_ref[1]


def _stats(eu):
    BN = 400
    nb = N // BN
    return pl.pallas_call(
        _stats_body,
        grid=(nb,),
        in_specs=[pl.BlockSpec((NC, BN, HALF), lambda i: (0, i, 0))],
        out_specs=[
            pl.BlockSpec((NC, HALF), lambda i: (0, 0)),
            pl.BlockSpec((NC, HALF), lambda i: (0, 0)),
        ],
        out_shape=[
            jax.ShapeDtypeStruct((NC, HALF), jnp.float32),
            jax.ShapeDtypeStruct((NC, HALF), jnp.float32),
        ],
        scratch_shapes=[pltpu.VMEM((2, NC, HALF), jnp.float32)],
    )(eu)


def _norm_body(eu_ref, hv_ref, sum_ref, sq_ref, g_ref, be_ref, out_ref):
    mean = sum_ref[...] / N
    var = sq_ref[...] / N - mean * mean
    inv = lax.rsqrt(var + EPS) * g_ref[...]
    for c in range(NC):
        hu = (eu_ref[c] - mean[c][None, :]) * inv[c][None, :] + be_ref[c][None, :]
        out_ref[:, c * HALF:(c + 1) * HALF] = (
            jnp.maximum(hu, 0.0) + hv_ref[:, c * HALF:(c + 1) * HALF]
        )


def _normalize(eu, hv, sums, sq, g2, be2):
    BN = 400
    nb = N // BN
    return pl.pallas_call(
        _norm_body,
        grid=(nb,),
        in_specs=[
            pl.BlockSpec((NC, BN, HALF), lambda i: (0, i, 0)),
            pl.BlockSpec((BN, D), lambda i: (i, 0)),
            pl.BlockSpec((NC, HALF), lambda i: (0, 0)),
            pl.BlockSpec((NC, HALF), lambda i: (0, 0)),
            pl.BlockSpec((NC, HALF), lambda i: (0, 0)),
            pl.BlockSpec((NC, HALF), lambda i: (0, 0)),
        ],
        out_specs=pl.BlockSpec((BN, D), lambda i: (i, 0)),
        out_shape=jax.ShapeDtypeStruct((N, D), jnp.float32),
    )(eu, hv, sums, sq, g2, be2)


def kernel(hv, edge_index, W, b, gamma, beta):
    W2 = W.reshape(R, D, NC, HALF).transpose(2, 0, 1, 3)
    b2 = b.reshape(R, NC, 1, HALF).transpose(1, 0, 2, 3)
    roff = jnp.arange(R, dtype=jnp.int32)[:, None] * N
    src_flat = (edge_index[:, 0, :] + roff).reshape(-1)
    dst_flat = edge_index[:, 1, :].reshape(-1)
    src_pad = jnp.concatenate(
        [src_flat, jnp.zeros((MP - M,), jnp.int32)]).reshape(NS, K, CH)
    dst_pad = jnp.concatenate(
        [dst_flat, jnp.full((MP - M,), N, jnp.int32)]).reshape(NS, K, CH)
    zeros = jnp.zeros((ROWS_PER_TILE, HALF), jnp.float32)

    table = _make_table(hv, W2, b2)
    eu = _sc_scatter(table, src_pad, dst_pad, zeros)
    sums, sq = _stats(eu)
    g2 = gamma.reshape(NC, HALF)
    be2 = beta.reshape(NC, HALF)
    return _normalize(eu, hv, sums, sq, g2, be2)


# speedup vs baseline: 2.1611x; 1.0982x over previous
import functools

import jax
import jax.numpy as jnp
from jax import lax
from jax.experimental import pallas as pl
from jax.experimental.pallas import tpu as pltpu
from jax.experimental.pallas import tpu_sc as plsc

N = 10000
E = 160000
R = 3
D = 256
EPS = 1e-5

NC = 2
NS = 16
CH = 128
HALF = D // 2
NP_NODES = 10240
ROWS_PER_TILE = NP_NODES // NS

M = R * E
KB = 40
K = -(-(-(-M // (NS * CH))) // KB) * KB
NB = K // KB
MP = NS * K * CH


def _mm_body(hv_ref, w_ref, b_ref, out_ref):
    out_ref[0] = (
        jnp.dot(hv_ref[...], w_ref[0, 0], preferred_element_type=jnp.float32)
        + b_ref[0, 0, 0][None, :]
    )


def _make_table(hv, W2, b2):
    BN = 400
    nb = N // BN
    return pl.pallas_call(
        _mm_body,
        grid=(NC, R, nb),
        in_specs=[
            pl.BlockSpec((BN, D), lambda c, r, i: (i, 0)),
            pl.BlockSpec((1, 1, D, HALF), lambda c, r, i: (c, r, 0, 0)),
            pl.BlockSpec((1, 1, 1, HALF), lambda c, r, i: (c, r, 0, 0)),
        ],
        out_specs=pl.BlockSpec((1, BN, HALF), lambda c, r, i: (c, r * nb + i, 0)),
        out_shape=jax.ShapeDtypeStruct((NC, R * N, HALF), jnp.float32),
    )(hv, W2, b2)


def _sc_body(table_hbm, src_hbm, dst_hbm, zeros_hbm, eu_hbm,
             src_v, dst_v, rows0, rows1, acc, g0, g1, s0, s1):
    c = lax.axis_index("c")
    s = lax.axis_index("s")
    pltpu.sync_copy(zeros_hbm, acc.at[pl.ds(s * ROWS_PER_TILE, ROWS_PER_TILE)])
    plsc.subcore_barrier()

    table = table_hbm.at[c]
    dummy = table.at[pl.ds(0, CH)]

    def outer(blk, carry):
        pltpu.sync_copy(src_hbm.at[s].at[pl.ds(blk * KB, KB)], src_v)
        pltpu.sync_copy(dst_hbm.at[s].at[pl.ds(blk * KB, KB)], dst_v)
        pltpu.async_copy(table.at[src_v.at[0]], rows0, g0)
        pltpu.async_copy(table.at[src_v.at[1]], rows1, g1)

        def pair(jj, carry2):
            j0 = 2 * jj
            j1 = j0 + 1
            pltpu.make_async_copy(dummy, rows0, g0).wait()
            d0 = pltpu.async_copy(rows0, acc.at[dst_v.at[j0]], s0, add=True)
            pltpu.make_async_copy(dummy, rows1, g1).wait()
            d1 = pltpu.async_copy(rows1, acc.at[dst_v.at[j1]], s1, add=True)
            d0.wait()

            @pl.when(jj < KB // 2 - 1)
            def _():
                pltpu.async_copy(table.at[src_v.at[j0 + 2]], rows0, g0)

            d1.wait()

            @pl.when(jj < KB // 2 - 1)
            def _():
                pltpu.async_copy(table.at[src_v.at[j1 + 2]], rows1, g1)

            return carry2

        return lax.fori_loop(0, KB // 2, pair, carry)

    lax.fori_loop(0, NB, outer, 0)
    plsc.subcore_barrier()
    pltpu.sync_copy(
        acc.at[pl.ds(s * ROWS_PER_TILE, ROWS_PER_TILE)],
        eu_hbm.at[c].at[pl.ds(s * ROWS_PER_TILE, ROWS_PER_TILE)],
    )


_sc_scatter = functools.partial(
    pl.kernel,
    out_type=jax.ShapeDtypeStruct((NC, NP_NODES, HALF), jnp.float32),
    mesh=plsc.VectorSubcoreMesh(core_axis_name="c", subcore_axis_name="s"),
    scratch_types=[
        pltpu.VMEM((KB, CH), jnp.int32),
        pltpu.VMEM((KB, CH), jnp.int32),
        pltpu.VMEM((CH, HALF), jnp.float32),
        pltpu.VMEM((CH, HALF), jnp.float32),
        pltpu.VMEM_SHARED((NP_NODES, HALF), jnp.float32),
        pltpu.SemaphoreType.DMA,
        pltpu.SemaphoreType.DMA,
        pltpu.SemaphoreType.DMA,
        pltpu.SemaphoreType.DMA,
    ],
)(_sc_body)


def _stats_body(eu_ref, sum_ref, sq_ref, acc_ref):
    i = pl.program_id(0)

    @pl.when(i == 0)
    def _():
        acc_ref[...] = jnp.zeros_like(acc_ref)

    x = eu_ref[...]
    acc_ref[0] += jnp.sum(x, axis=1)
    acc_ref[1] += jnp.sum(x * x, axis=1)

    @pl.when(i == pl.num_programs(0) - 1)
    def _():
        sum_ref[...] = acc_ref[0]
        sq_ref[...] = acc_ref[1]


def _stats(eu):
    BN = 400
    nb = N // BN
    return pl.pallas_call(
        _stats_body,
        grid=(nb,),
        in_specs=[pl.BlockSpec((NC, BN, HALF), lambda i: (0, i, 0))],
        out_specs=[
            pl.BlockSpec((NC, HALF), lambda i: (0, 0)),
            pl.BlockSpec((NC, HALF), lambda i: (0, 0)),
        ],
        out_shape=[
            jax.ShapeDtypeStruct((NC, HALF), jnp.float32),
            jax.ShapeDtypeStruct((NC, HALF), jnp.float32),
        ],
        scratch_shapes=[pltpu.VMEM((2, NC, HALF), jnp.float32)],
    )(eu)


def _norm_body(eu_ref, hv_ref, sum_ref, sq_ref, g_ref, be_ref, out_ref):
    mean = sum_ref[...] / N
    var = sq_ref[...] / N - mean * mean
    inv = lax.rsqrt(var + EPS) * g_ref[...]
    for c in range(NC):
        hu = (eu_ref[c] - mean[c][None, :]) * inv[c][None, :] + be_ref[c][None, :]
        out_ref[:, c * HALF:(c + 1) * HALF] = (
            jnp.maximum(hu, 0.0) + hv_ref[:, c * HALF:(c + 1) * HALF]
        )


def _normalize(eu, hv, sums, sq, g2, be2):
    BN = 400
    nb = N // BN
    return pl.pallas_call(
        _norm_body,
        grid=(nb,),
        in_specs=[
            pl.BlockSpec((NC, BN, HALF), lambda i: (0, i, 0)),
            pl.BlockSpec((BN, D), lambda i: (i, 0)),
            pl.BlockSpec((NC, HALF), lambda i: (0, 0)),
            pl.BlockSpec((NC, HALF), lambda i: (0, 0)),
            pl.BlockSpec((NC, HALF), lambda i: (0, 0)),
            pl.BlockSpec((NC, HALF), lambda i: (0, 0)),
        ],
        out_specs=pl.BlockSpec((BN, D), lambda i: (i, 0)),
        out_shape=jax.ShapeDtypeStruct((N, D), jnp.float32),
    )(eu, hv, sums, sq, g2, be2)


def kernel(hv, edge_index, W, b, gamma, beta):
    W2 = W.reshape(R, D, NC, HALF).transpose(2, 0, 1, 3)
    b2 = b.reshape(R, NC, 1, HALF).transpose(1, 0, 2, 3)
    roff = jnp.arange(R, dtype=jnp.int32)[:, None] * N
    src_flat = (edge_index[:, 0, :] + roff).reshape(-1)
    dst_flat = edge_index[:, 1, :].reshape(-1)
    src_pad = jnp.concatenate(
        [src_flat, jnp.zeros((MP - M,), jnp.int32)]).reshape(NS, K, CH)
    dst_pad = jnp.concatenate(
        [dst_flat, jnp.full((MP - M,), N, jnp.int32)]).reshape(NS, K, CH)
    zeros = jnp.zeros((ROWS_PER_TILE, HALF), jnp.float32)

    table = _make_table(hv, W2, b2)
    eu = _sc_scatter(table, src_pad, dst_pad, zeros)
    sums, sq = _stats(eu)
    g2 = gamma.reshape(NC, HALF)
    be2 = beta.reshape(NC, HALF)
    return _normalize(eu, hv, sums, sq, g2, be2)


# speedup vs baseline: 2.1672x; 1.0028x over previous
import functools

import jax
import jax.numpy as jnp
from jax import lax
from jax.experimental import pallas as pl
from jax.experimental.pallas import tpu as pltpu
from jax.experimental.pallas import tpu_sc as plsc

N = 10000
E = 160000
R = 3
D = 256
EPS = 1e-5

NC = 2
NS = 16
CH = 128
HALF = D // 2
NP_NODES = 10240
ROWS_PER_TILE = NP_NODES // NS

M = R * E
KB = 40
K = -(-(-(-M // (NS * CH))) // KB) * KB
NB = K // KB
MP = NS * K * CH


def _mm_body(hv_ref, w_ref, b_ref, out_ref):
    out_ref[0] = (
        jnp.dot(hv_ref[...], w_ref[0, 0], preferred_element_type=jnp.float32)
        + b_ref[0, 0, 0][None, :]
    )


def _make_table(hv, W2, b2):
    BN = 400
    nb = N // BN
    return pl.pallas_call(
        _mm_body,
        grid=(NC, R, nb),
        in_specs=[
            pl.BlockSpec((BN, D), lambda c, r, i: (i, 0)),
            pl.BlockSpec((1, 1, D, HALF), lambda c, r, i: (c, r, 0, 0)),
            pl.BlockSpec((1, 1, 1, HALF), lambda c, r, i: (c, r, 0, 0)),
        ],
        out_specs=pl.BlockSpec((1, BN, HALF), lambda c, r, i: (c, r * nb + i, 0)),
        out_shape=jax.ShapeDtypeStruct((NC, R * N, HALF), jnp.float32),
    )(hv, W2, b2)


def _sc_body(table_hbm, src_hbm, dst_hbm, zeros_hbm, eu_hbm,
             src_v, dst_v, rows0, rows1, acc, g0, g1, s0, s1):
    c = lax.axis_index("c")
    s = lax.axis_index("s")
    pltpu.sync_copy(zeros_hbm, acc.at[pl.ds(s * ROWS_PER_TILE, ROWS_PER_TILE)])
    plsc.subcore_barrier()

    table = table_hbm.at[c]
    dummy = table.at[pl.ds(0, CH)]

    def outer(blk, carry):
        pltpu.sync_copy(src_hbm.at[s].at[pl.ds(blk * KB, KB)], src_v)
        pltpu.sync_copy(dst_hbm.at[s].at[pl.ds(blk * KB, KB)], dst_v)

        def fire_gather(j, buf, sem):
            pltpu.async_copy(
                table.at[src_v.at[j, pl.ds(0, CH // 2)]],
                buf.at[pl.ds(0, CH // 2)], sem)
            pltpu.async_copy(
                table.at[src_v.at[j, pl.ds(CH // 2, CH // 2)]],
                buf.at[pl.ds(CH // 2, CH // 2)], sem)

        fire_gather(0, rows0, g0)
        fire_gather(1, rows1, g1)

        def pair(jj, carry2):
            j0 = 2 * jj
            j1 = j0 + 1
            pltpu.make_async_copy(dummy, rows0, g0).wait()
            d0 = pltpu.async_copy(rows0, acc.at[dst_v.at[j0]], s0, add=True)
            pltpu.make_async_copy(dummy, rows1, g1).wait()
            d1 = pltpu.async_copy(rows1, acc.at[dst_v.at[j1]], s1, add=True)
            d0.wait()

            @pl.when(jj < KB // 2 - 1)
            def _():
                fire_gather(j0 + 2, rows0, g0)

            d1.wait()

            @pl.when(jj < KB // 2 - 1)
            def _():
                fire_gather(j1 + 2, rows1, g1)

            return carry2

        return lax.fori_loop(0, KB // 2, pair, carry)

    lax.fori_loop(0, NB, outer, 0)
    plsc.subcore_barrier()
    pltpu.sync_copy(
        acc.at[pl.ds(s * ROWS_PER_TILE, ROWS_PER_TILE)],
        eu_hbm.at[c].at[pl.ds(s * ROWS_PER_TILE, ROWS_PER_TILE)],
    )


_sc_scatter = functools.partial(
    pl.kernel,
    out_type=jax.ShapeDtypeStruct((NC, NP_NODES, HALF), jnp.float32),
    mesh=plsc.VectorSubcoreMesh(core_axis_name="c", subcore_axis_name="s"),
    scratch_types=[
        pltpu.VMEM((KB, CH), jnp.int32),
        pltpu.VMEM((KB, CH), jnp.int32),
        pltpu.VMEM((CH, HALF), jnp.float32),
        pltpu.VMEM((CH, HALF), jnp.float32),
        pltpu.VMEM_SHARED((NP_NODES, HALF), jnp.float32),
        pltpu.SemaphoreType.DMA,
        pltpu.SemaphoreType.DMA,
        pltpu.SemaphoreType.DMA,
        pltpu.SemaphoreType.DMA,
    ],
)(_sc_body)


def _stats_body(eu_ref, sum_ref, sq_ref, acc_ref):
    i = pl.program_id(0)

    @pl.when(i == 0)
    def _():
        acc_ref[...] = jnp.zeros_like(acc_ref)

    x = eu_ref[...]
    acc_ref[0] += jnp.sum(x, axis=1)
    acc_ref[1] += jnp.sum(x * x, axis=1)

    @pl.when(i == pl.num_programs(0) - 1)
    def _():
        sum_ref[...] = acc_ref[0]
        sq_ref[...] = acc_ref[1]


def _stats(eu):
    BN = 400
    nb = N // BN
    return pl.pallas_call(
        _stats_body,
        grid=(nb,),
        in_specs=[pl.BlockSpec((NC, BN, HALF), lambda i: (0, i, 0))],
        out_specs=[
            pl.BlockSpec((NC, HALF), lambda i: (0, 0)),
            pl.BlockSpec((NC, HALF), lambda i: (0, 0)),
        ],
        out_shape=[
            jax.ShapeDtypeStruct((NC, HALF), jnp.float32),
            jax.ShapeDtypeStruct((NC, HALF), jnp.float32),
        ],
        scratch_shapes=[pltpu.VMEM((2, NC, HALF), jnp.float32)],
    )(eu)


def _norm_body(eu_ref, hv_ref, sum_ref, sq_ref, g_ref, be_ref, out_ref):
    mean = sum_ref[...] / N
    var = sq_ref[...] / N - mean * mean
    inv = lax.rsqrt(var + EPS) * g_ref[...]
    for c in range(NC):
        hu = (eu_ref[c] - mean[c][None, :]) * inv[c][None, :] + be_ref[c][None, :]
        out_ref[:, c * HALF:(c + 1) * HALF] = (
            jnp.maximum(hu, 0.0) + hv_ref[:, c * HALF:(c + 1) * HALF]
        )


def _normalize(eu, hv, sums, sq, g2, be2):
    BN = 400
    nb = N // BN
    return pl.pallas_call(
        _norm_body,
        grid=(nb,),
        in_specs=[
            pl.BlockSpec((NC, BN, HALF), lambda i: (0, i, 0)),
            pl.BlockSpec((BN, D), lambda i: (i, 0)),
            pl.BlockSpec((NC, HALF), lambda i: (0, 0)),
            pl.BlockSpec((NC, HALF), lambda i: (0, 0)),
            pl.BlockSpec((NC, HALF), lambda i: (0, 0)),
            pl.BlockSpec((NC, HALF), lambda i: (0, 0)),
        ],
        out_specs=pl.BlockSpec((BN, D), lambda i: (i, 0)),
        out_shape=jax.ShapeDtypeStruct((N, D), jnp.float32),
    )(eu, hv, sums, sq, g2, be2)


def kernel(hv, edge_index, W, b, gamma, beta):
    W2 = W.reshape(R, D, NC, HALF).transpose(2, 0, 1, 3)
    b2 = b.reshape(R, NC, 1, HALF).transpose(1, 0, 2, 3)
    roff = jnp.arange(R, dtype=jnp.int32)[:, None] * N
    src_flat = (edge_index[:, 0, :] + roff).reshape(-1)
    dst_flat = edge_index[:, 1, :].reshape(-1)
    src_pad = jnp.concatenate(
        [src_flat, jnp.zeros((MP - M,), jnp.int32)]).reshape(NS, K, CH)
    dst_pad = jnp.concatenate(
        [dst_flat, jnp.full((MP - M,), N, jnp.int32)]).reshape(NS, K, CH)
    zeros = jnp.zeros((ROWS_PER_TILE, HALF), jnp.float32)

    table = _make_table(hv, W2, b2)
    eu = _sc_scatter(table, src_pad, dst_pad, zeros)
    sums, sq = _stats(eu)
    g2 = gamma.reshape(NC, HALF)
    be2 = beta.reshape(NC, HALF)
    return _normalize(eu, hv, sums, sq, g2, be2)


# speedup vs baseline: 2.3084x; 1.0652x over previous
import functools

import jax
import jax.numpy as jnp
from jax import lax
from jax.experimental import pallas as pl
from jax.experimental.pallas import tpu as pltpu
from jax.experimental.pallas import tpu_sc as plsc

N = 10000
E = 160000
R = 3
D = 256
EPS = 1e-5

NC = 2
NS = 16
CH = 128
HALF = D // 2
NP_NODES = 10240
ROWS_PER_TILE = NP_NODES // NS

M = R * E
KB = 48
K = -(-(-(-M // (NS * CH))) // KB) * KB
NB = K // KB
MP = NS * K * CH


def _mm_body(hv_ref, w_ref, b_ref, out_ref):
    out_ref[0] = (
        jnp.dot(hv_ref[...], w_ref[0, 0], preferred_element_type=jnp.float32)
        + b_ref[0, 0, 0][None, :]
    )


def _make_table(hv, W2, b2):
    BN = 400
    nb = N // BN
    return pl.pallas_call(
        _mm_body,
        grid=(NC, R, nb),
        in_specs=[
            pl.BlockSpec((BN, D), lambda c, r, i: (i, 0)),
            pl.BlockSpec((1, 1, D, HALF), lambda c, r, i: (c, r, 0, 0)),
            pl.BlockSpec((1, 1, 1, HALF), lambda c, r, i: (c, r, 0, 0)),
        ],
        out_specs=pl.BlockSpec((1, BN, HALF), lambda c, r, i: (c, r * nb + i, 0)),
        out_shape=jax.ShapeDtypeStruct((NC, R * N, HALF), jnp.float32),
    )(hv, W2, b2)


def _sc_body(table_hbm, src_hbm, dst_hbm, zeros_hbm, eu_hbm,
             src_v, dst_v, rows0, rows1, acc, g0, g1, s0, s1):
    c = 1 - lax.axis_index("c")
    s = lax.axis_index("s")
    pltpu.sync_copy(zeros_hbm, acc.at[pl.ds(s * ROWS_PER_TILE, ROWS_PER_TILE)])
    plsc.subcore_barrier()

    table = table_hbm.at[c]
    dummy = table.at[pl.ds(0, CH)]

    def outer(blk, carry):
        pltpu.sync_copy(src_hbm.at[s].at[pl.ds(blk * KB, KB)], src_v)
        pltpu.sync_copy(dst_hbm.at[s].at[pl.ds(blk * KB, KB)], dst_v)

        def fire_gather(j, buf, sem):
            pltpu.async_copy(table.at[src_v.at[j]], buf, sem)

        fire_gather(0, rows0, g0)
        fire_gather(1, rows1, g1)

        def pair(jj, carry2):
            j0 = 2 * jj
            j1 = j0 + 1
            pltpu.make_async_copy(dummy, rows0, g0).wait()
            d0 = pltpu.async_copy(rows0, acc.at[dst_v.at[j0]], s0, add=True)
            pltpu.make_async_copy(dummy, rows1, g1).wait()
            d1 = pltpu.async_copy(rows1, acc.at[dst_v.at[j1]], s1, add=True)
            d0.wait()

            @pl.when(jj < KB // 2 - 1)
            def _():
                fire_gather(j0 + 2, rows0, g0)

            d1.wait()

            @pl.when(jj < KB // 2 - 1)
            def _():
                fire_gather(j1 + 2, rows1, g1)

            return carry2

        return lax.fori_loop(0, KB // 2, pair, carry)

    lax.fori_loop(0, NB, outer, 0)
    plsc.subcore_barrier()
    pltpu.sync_copy(
        acc.at[pl.ds(s * ROWS_PER_TILE, ROWS_PER_TILE)],
        eu_hbm.at[c].at[pl.ds(s * ROWS_PER_TILE, ROWS_PER_TILE)],
    )


_sc_scatter = functools.partial(
    pl.kernel,
    out_type=jax.ShapeDtypeStruct((NC, NP_NODES, HALF), jnp.float32),
    mesh=plsc.VectorSubcoreMesh(core_axis_name="c", subcore_axis_name="s"),
    scratch_types=[
        pltpu.VMEM((KB, CH), jnp.int32),
        pltpu.VMEM((KB, CH), jnp.int32),
        pltpu.VMEM((CH, HALF), jnp.float32),
        pltpu.VMEM((CH, HALF), jnp.float32),
        pltpu.VMEM_SHARED((NP_NODES, HALF), jnp.float32),
        pltpu.SemaphoreType.DMA,
        pltpu.SemaphoreType.DMA,
        pltpu.SemaphoreType.DMA,
        pltpu.SemaphoreType.DMA,
    ],
)(_sc_body)


def _bn_body(eu_ref, hv_ref, g_ref, be_ref, out_ref, acc_ref):
    p = pl.program_id(0)
    i = pl.program_id(1)

    @pl.when((p == 0) & (i == 0))
    def _():
        acc_ref[...] = jnp.zeros_like(acc_ref)

    @pl.when(p == 0)
    def _():
        x = eu_ref[...]
        acc_ref[0] += jnp.sum(x, axis=1)
        acc_ref[1] += jnp.sum(x * x, axis=1)

    @pl.when(p == 1)
    def _():
        mean = acc_ref[0] / N
        var = acc_ref[1] / N - mean * mean
        inv = lax.rsqrt(var + EPS) * g_ref[...]
        for c in range(NC):
            hu = ((eu_ref[c] - mean[c][None, :]) * inv[c][None, :]
                  + be_ref[c][None, :])
            out_ref[:, c * HALF:(c + 1) * HALF] = (
                jnp.maximum(hu, 0.0) + hv_ref[:, c * HALF:(c + 1) * HALF]
            )


def _batchnorm(eu, hv, g2, be2):
    BN = 400
    nb = N // BN
    return pl.pallas_call(
        _bn_body,
        grid=(2, nb),
        in_specs=[
            pl.BlockSpec((NC, BN, HALF), lambda p, i: (0, i, 0)),
            pl.BlockSpec((BN, D), lambda p, i: (i, 0)),
            pl.BlockSpec((NC, HALF), lambda p, i: (0, 0)),
            pl.BlockSpec((NC, HALF), lambda p, i: (0, 0)),
        ],
        out_specs=pl.BlockSpec((BN, D), lambda p, i: (i, 0)),
        out_shape=jax.ShapeDtypeStruct((N, D), jnp.float32),
        scratch_shapes=[pltpu.VMEM((2, NC, HALF), jnp.float32)],
    )(eu, hv, g2, be2)


def kernel(hv, edge_index, W, b, gamma, beta):
    W2 = W.reshape(R, D, NC, HALF).transpose(2, 0, 1, 3)
    b2 = b.reshape(R, NC, 1, HALF).transpose(1, 0, 2, 3)
    roff = jnp.arange(R, dtype=jnp.int32)[:, None] * N
    src_flat = (edge_index[:, 0, :] + roff).reshape(-1)
    dst_flat = edge_index[:, 1, :].reshape(-1)
    src_pad = jnp.concatenate(
        [src_flat, jnp.zeros((MP - M,), jnp.int32)]).reshape(NS, K, CH)
    dst_pad = jnp.concatenate(
        [dst_flat, jnp.full((MP - M,), N, jnp.int32)]).reshape(NS, K, CH)
    zeros = jnp.zeros((ROWS_PER_TILE, HALF), jnp.float32)

    table = _make_table(hv, W2, b2)
    eu = _sc_scatter(table, src_pad, dst_pad, zeros)
    g2 = gamma.reshape(NC, HALF)
    be2 = beta.reshape(NC, HALF)
    return _batchnorm(eu, hv, g2, be2)


# speedup vs baseline: 2.3220x; 1.0059x over previous
import functools

import jax
import jax.numpy as jnp
from jax import lax
from jax.experimental import pallas as pl
from jax.experimental.pallas import tpu as pltpu
from jax.experimental.pallas import tpu_sc as plsc

N = 10000
E = 160000
R = 3
D = 256
EPS = 1e-5

NC = 2
NS = 16
CH = 128
HALF = D // 2
NP_NODES = 10240
ROWS_PER_TILE = NP_NODES // NS

M = R * E
KB = 24
K = -(-(-(-M // (NS * CH))) // KB) * KB
NB = K // KB
MP = NS * K * CH


def _mm_body(hv_ref, w_ref, b_ref, out_ref):
    out_ref[0] = (
        jnp.dot(hv_ref[...], w_ref[0, 0], preferred_element_type=jnp.float32)
        + b_ref[0, 0, 0][None, :]
    )


def _make_table(hv, W2, b2):
    BN = 400
    nb = N // BN
    return pl.pallas_call(
        _mm_body,
        grid=(NC, R, nb),
        in_specs=[
            pl.BlockSpec((BN, D), lambda c, r, i: (i, 0)),
            pl.BlockSpec((1, 1, D, HALF), lambda c, r, i: (c, r, 0, 0)),
            pl.BlockSpec((1, 1, 1, HALF), lambda c, r, i: (c, r, 0, 0)),
        ],
        out_specs=pl.BlockSpec((1, BN, HALF), lambda c, r, i: (c, r * nb + i, 0)),
        out_shape=jax.ShapeDtypeStruct((NC, R * N, HALF), jnp.float32),
    )(hv, W2, b2)


NSB = NB // 2


def _sc_body(table_hbm, src_hbm, dst_hbm, zeros_hbm, eu_hbm,
             src_a, dst_a, src_b, dst_b, rows0, rows1, acc,
             g0, g1, s0, s1, ia, ib):
    c = 1 - lax.axis_index("c")
    s = lax.axis_index("s")
    pltpu.sync_copy(zeros_hbm, acc.at[pl.ds(s * ROWS_PER_TILE, ROWS_PER_TILE)])
    plsc.subcore_barrier()

    table = table_hbm.at[c]
    dummy = table.at[pl.ds(0, CH)]
    idummy = src_hbm.at[s].at[pl.ds(0, KB)]
    src_slab = src_hbm.at[s]
    dst_slab = dst_hbm.at[s]
    LAST = KB // 2 - 1

    def stage(blk, sv, dv, sem):
        pltpu.async_copy(src_slab.at[pl.ds(blk * KB, KB)], sv, sem)
        pltpu.async_copy(dst_slab.at[pl.ds(blk * KB, KB)], dv, sem)

    def wait_stage(sv, dv, sem):
        pltpu.make_async_copy(idummy, sv, sem).wait()
        pltpu.make_async_copy(idummy, dv, sem).wait()

    def fire(sv, j, buf, sem):
        pltpu.async_copy(table.at[sv.at[j]], buf, sem)

    pltpu.sync_copy(src_slab.at[pl.ds(0, KB)], src_a)
    pltpu.sync_copy(dst_slab.at[pl.ds(0, KB)], dst_a)
    stage(1, src_b, dst_b, ib)
    fire(src_a, 0, rows0, g0)
    fire(src_a, 1, rows1, g1)

    def block_pairs(sv, dv, nsv, ndv, stage_sem, has_next):
        def pair(p, carry):
            j0 = 2 * p
            j1 = j0 + 1
            pltpu.make_async_copy(dummy, rows0, g0).wait()
            d0 = pltpu.async_copy(rows0, acc.at[dv.at[j0]], s0, add=True)
            pltpu.make_async_copy(dummy, rows1, g1).wait()
            d1 = pltpu.async_copy(rows1, acc.at[dv.at[j1]], s1, add=True)
            d0.wait()

            @pl.when(p < LAST)
            def _():
                fire(sv, j0 + 2, rows0, g0)

            @pl.when((p == LAST) & has_next)
            def _():
                wait_stage(nsv, ndv, stage_sem)
                fire(nsv, 0, rows0, g0)

            d1.wait()

            @pl.when(p < LAST)
            def _():
                fire(sv, j1 + 2, rows1, g1)

            @pl.when((p == LAST) & has_next)
            def _():
                fire(nsv, 1, rows1, g1)

            return carry

        return lax.fori_loop(0, KB // 2, pair, 0)

    def superblock(sb, carry):
        true_ = sb >= 0
        block_pairs(src_a, dst_a, src_b, dst_b, ib, true_)
        @pl.when(sb < NSB - 1)
        def _():
            stage(2 * sb + 2, src_a, dst_a, ia)

        block_pairs(src_b, dst_b, src_a, dst_a, ia, sb < NSB - 1)

        @pl.when(sb < NSB - 1)
        def _():
            stage(2 * sb + 3, src_b, dst_b, ib)

        return carry

    lax.fori_loop(0, NSB, superblock, 0)
    plsc.subcore_barrier()
    pltpu.sync_copy(
        acc.at[pl.ds(s * ROWS_PER_TILE, ROWS_PER_TILE)],
        eu_hbm.at[c].at[pl.ds(s * ROWS_PER_TILE, ROWS_PER_TILE)],
    )


_sc_scatter = functools.partial(
    pl.kernel,
    out_type=jax.ShapeDtypeStruct((NC, NP_NODES, HALF), jnp.float32),
    mesh=plsc.VectorSubcoreMesh(core_axis_name="c", subcore_axis_name="s"),
    scratch_types=[
        pltpu.VMEM((KB, CH), jnp.int32),
        pltpu.VMEM((KB, CH), jnp.int32),
        pltpu.VMEM((KB, CH), jnp.int32),
        pltpu.VMEM((KB, CH), jnp.int32),
        pltpu.VMEM((CH, HALF), jnp.float32),
        pltpu.VMEM((CH, HALF), jnp.float32),
        pltpu.VMEM_SHARED((NP_NODES, HALF), jnp.float32),
        pltpu.SemaphoreType.DMA,
        pltpu.SemaphoreType.DMA,
        pltpu.SemaphoreType.DMA,
        pltpu.SemaphoreType.DMA,
        pltpu.SemaphoreType.DMA,
        pltpu.SemaphoreType.DMA,
    ],
)(_sc_body)


def _bn_body(eu_ref, hv_ref, g_ref, be_ref, out_ref, acc_ref):
    p = pl.program_id(0)
    i = pl.program_id(1)

    @pl.when((p == 0) & (i == 0))
    def _():
        acc_ref[...] = jnp.zeros_like(acc_ref)

    @pl.when(p == 0)
    def _():
        x = eu_ref[...]
        acc_ref[0] += jnp.sum(x, axis=1)
        acc_ref[1] += jnp.sum(x * x, axis=1)

    @pl.when(p == 1)
    def _():
        mean = acc_ref[0] / N
        var = acc_ref[1] / N - mean * mean
        inv = lax.rsqrt(var + EPS) * g_ref[...]
        for c in range(NC):
            hu = ((eu_ref[c] - mean[c][None, :]) * inv[c][None, :]
                  + be_ref[c][None, :])
            out_ref[:, c * HALF:(c + 1) * HALF] = (
                jnp.maximum(hu, 0.0) + hv_ref[:, c * HALF:(c + 1) * HALF]
            )


def _batchnorm(eu, hv, g2, be2):
    BN = 400
    nb = N // BN
    return pl.pallas_call(
        _bn_body,
        grid=(2, nb),
        in_specs=[
            pl.BlockSpec((NC, BN, HALF), lambda p, i: (0, i, 0)),
            pl.BlockSpec((BN, D), lambda p, i: (i, 0)),
            pl.BlockSpec((NC, HALF), lambda p, i: (0, 0)),
            pl.BlockSpec((NC, HALF), lambda p, i: (0, 0)),
        ],
        out_specs=pl.BlockSpec((BN, D), lambda p, i: (i, 0)),
        out_shape=jax.ShapeDtypeStruct((N, D), jnp.float32),
        scratch_shapes=[pltpu.VMEM((2, NC, HALF), jnp.float32)],
    )(eu, hv, g2, be2)


def kernel(hv, edge_index, W, b, gamma, beta):
    W2 = W.reshape(R, D, NC, HALF).transpose(2, 0, 1, 3)
    b2 = b.reshape(R, NC, 1, HALF).transpose(1, 0, 2, 3)
    roff = jnp.arange(R, dtype=jnp.int32)[:, None] * N
    src_flat = (edge_index[:, 0, :] + roff).reshape(-1)
    dst_flat = edge_index[:, 1, :].reshape(-1)
    src_pad = jnp.concatenate(
        [src_flat, jnp.zeros((MP - M,), jnp.int32)]).reshape(NS, K, CH)
    dst_pad = jnp.concatenate(
        [dst_flat, jnp.full((MP - M,), N, jnp.int32)]).reshape(NS, K, CH)
    zeros = jnp.zeros((ROWS_PER_TILE, HALF), jnp.float32)

    table = _make_table(hv, W2, b2)
    eu = _sc_scatter(table, src_pad, dst_pad, zeros)
    g2 = gamma.reshape(NC, HALF)
    be2 = beta.reshape(NC, HALF)
    return _batchnorm(eu, hv, g2, be2)


# speedup vs baseline: 2.3236x; 1.0007x over previous
import functools

import jax
import jax.numpy as jnp
from jax import lax
from jax.experimental import pallas as pl
from jax.experimental.pallas import tpu as pltpu
from jax.experimental.pallas import tpu_sc as plsc

N = 10000
E = 160000
R = 3
D = 256
EPS = 1e-5

NC = 2
NS = 16
CH = 128
HALF = D // 2
NP_NODES = 10240
ROWS_PER_TILE = NP_NODES // NS

M = R * E
KB = 24
K = -(-(-(-M // (NS * CH))) // KB) * KB
NB = K // KB
MP = NS * K * CH


def _mm_body(hv_ref, w_ref, b_ref, out_ref):
    out_ref[0] = (
        jnp.dot(hv_ref[...], w_ref[0, 0], preferred_element_type=jnp.float32)
        + b_ref[0, 0, 0][None, :]
    )


def _make_table(hv, W2, b2):
    BN = 400
    nb = N // BN
    return pl.pallas_call(
        _mm_body,
        grid=(NC, R, nb),
        in_specs=[
            pl.BlockSpec((BN, D), lambda c, r, i: (i, 0)),
            pl.BlockSpec((1, 1, D, HALF), lambda c, r, i: (c, r, 0, 0)),
            pl.BlockSpec((1, 1, 1, HALF), lambda c, r, i: (c, r, 0, 0)),
        ],
        out_specs=pl.BlockSpec((1, BN, HALF), lambda c, r, i: (c, r * nb + i, 0)),
        out_shape=jax.ShapeDtypeStruct((NC, R * N, HALF), jnp.float32),
    )(hv, W2, b2)


NSB = NB // 2


def _sc_body(table_hbm, src_hbm, dst_hbm, zeros_hbm, eu_hbm,
             src_a, dst_a, src_b, dst_b, rows0, rows1, acc,
             g0, g1, s0, s1, ia, ib):
    c = 1 - lax.axis_index("c")
    s = lax.axis_index("s")
    table = table_hbm.at[c]
    dummy = table.at[pl.ds(0, CH)]
    idummy = src_hbm.at[s].at[pl.ds(0, KB)]
    src_slab = src_hbm.at[s]
    dst_slab = dst_hbm.at[s]
    LAST = KB // 2 - 1

    def stage(blk, sv, dv, sem):
        pltpu.async_copy(src_slab.at[pl.ds(blk * KB, KB)], sv, sem)
        pltpu.async_copy(dst_slab.at[pl.ds(blk * KB, KB)], dv, sem)

    def wait_stage(sv, dv, sem):
        pltpu.make_async_copy(idummy, sv, sem).wait()
        pltpu.make_async_copy(idummy, dv, sem).wait()

    def fire(sv, j, buf, sem):
        pltpu.async_copy(table.at[sv.at[j]], buf, sem)

    stage(0, src_a, dst_a, ia)
    stage(1, src_b, dst_b, ib)
    pltpu.sync_copy(zeros_hbm, rows0)
    for k in range(ROWS_PER_TILE // CH):
        pltpu.sync_copy(rows0, acc.at[pl.ds(s * ROWS_PER_TILE + k * CH, CH)])
    plsc.subcore_barrier()
    wait_stage(src_a, dst_a, ia)
    fire(src_a, 0, rows0, g0)
    fire(src_a, 1, rows1, g1)

    def block_pairs(sv, dv, nsv, ndv, stage_sem, has_next):
        def pair(p, carry):
            j0 = 2 * p
            j1 = j0 + 1
            pltpu.make_async_copy(dummy, rows0, g0).wait()
            d0 = pltpu.async_copy(rows0, acc.at[dv.at[j0]], s0, add=True)
            pltpu.make_async_copy(dummy, rows1, g1).wait()
            d1 = pltpu.async_copy(rows1, acc.at[dv.at[j1]], s1, add=True)
            d0.wait()

            @pl.when(p < LAST)
            def _():
                fire(sv, j0 + 2, rows0, g0)

            @pl.when((p == LAST) & has_next)
            def _():
                wait_stage(nsv, ndv, stage_sem)
                fire(nsv, 0, rows0, g0)

            d1.wait()

            @pl.when(p < LAST)
            def _():
                fire(sv, j1 + 2, rows1, g1)

            @pl.when((p == LAST) & has_next)
            def _():
                fire(nsv, 1, rows1, g1)

            return carry

        return lax.fori_loop(0, KB // 2, pair, 0)

    def superblock(sb, carry):
        true_ = sb >= 0
        block_pairs(src_a, dst_a, src_b, dst_b, ib, true_)
        @pl.when(sb < NSB - 1)
        def _():
            stage(2 * sb + 2, src_a, dst_a, ia)

        block_pairs(src_b, dst_b, src_a, dst_a, ia, sb < NSB - 1)

        @pl.when(sb < NSB - 1)
        def _():
            stage(2 * sb + 3, src_b, dst_b, ib)

        return carry

    lax.fori_loop(0, NSB, superblock, 0)
    plsc.subcore_barrier()
    pltpu.sync_copy(
        acc.at[pl.ds(s * ROWS_PER_TILE, ROWS_PER_TILE)],
        eu_hbm.at[c].at[pl.ds(s * ROWS_PER_TILE, ROWS_PER_TILE)],
    )


_sc_scatter = functools.partial(
    pl.kernel,
    out_type=jax.ShapeDtypeStruct((NC, NP_NODES, HALF), jnp.float32),
    mesh=plsc.VectorSubcoreMesh(core_axis_name="c", subcore_axis_name="s"),
    scratch_types=[
        pltpu.VMEM((KB, CH), jnp.int32),
        pltpu.VMEM((KB, CH), jnp.int32),
        pltpu.VMEM((KB, CH), jnp.int32),
        pltpu.VMEM((KB, CH), jnp.int32),
        pltpu.VMEM((CH, HALF), jnp.float32),
        pltpu.VMEM((CH, HALF), jnp.float32),
        pltpu.VMEM_SHARED((NP_NODES, HALF), jnp.float32),
        pltpu.SemaphoreType.DMA,
        pltpu.SemaphoreType.DMA,
        pltpu.SemaphoreType.DMA,
        pltpu.SemaphoreType.DMA,
        pltpu.SemaphoreType.DMA,
        pltpu.SemaphoreType.DMA,
    ],
)(_sc_body)


def _bn_body(eu_ref, hv_ref, g_ref, be_ref, out_ref, acc_ref):
    p = pl.program_id(0)
    i = pl.program_id(1)

    @pl.when((p == 0) & (i == 0))
    def _():
        acc_ref[...] = jnp.zeros_like(acc_ref)

    @pl.when(p == 0)
    def _():
        x = eu_ref[...]
        acc_ref[0] += jnp.sum(x, axis=1)
        acc_ref[1] += jnp.sum(x * x, axis=1)

    @pl.when(p == 1)
    def _():
        mean = acc_ref[0] / N
        var = acc_ref[1] / N - mean * mean
        inv = lax.rsqrt(var + EPS) * g_ref[...]
        for c in range(NC):
            hu = ((eu_ref[c] - mean[c][None, :]) * inv[c][None, :]
                  + be_ref[c][None, :])
            out_ref[:, c * HALF:(c + 1) * HALF] = (
                jnp.maximum(hu, 0.0) + hv_ref[:, c * HALF:(c + 1) * HALF]
            )


def _batchnorm(eu, hv, g2, be2):
    BN = 400
    nb = N // BN
    return pl.pallas_call(
        _bn_body,
        grid=(2, nb),
        in_specs=[
            pl.BlockSpec((NC, BN, HALF), lambda p, i: (0, i, 0)),
            pl.BlockSpec((BN, D), lambda p, i: (i, 0)),
            pl.BlockSpec((NC, HALF), lambda p, i: (0, 0)),
            pl.BlockSpec((NC, HALF), lambda p, i: (0, 0)),
        ],
        out_specs=pl.BlockSpec((BN, D), lambda p, i: (i, 0)),
        out_shape=jax.ShapeDtypeStruct((N, D), jnp.float32),
        scratch_shapes=[pltpu.VMEM((2, NC, HALF), jnp.float32)],
    )(eu, hv, g2, be2)


def kernel(hv, edge_index, W, b, gamma, beta):
    W2 = W.reshape(R, D, NC, HALF).transpose(2, 0, 1, 3)
    b2 = b.reshape(R, NC, 1, HALF).transpose(1, 0, 2, 3)
    roff = jnp.arange(R, dtype=jnp.int32)[:, None] * N
    src_flat = (edge_index[:, 0, :] + roff).reshape(-1)
    dst_flat = edge_index[:, 1, :].reshape(-1)
    src_pad = jnp.concatenate(
        [src_flat, jnp.zeros((MP - M,), jnp.int32)]).reshape(NS, K, CH)
    dst_pad = jnp.concatenate(
        [dst_flat, jnp.full((MP - M,), N, jnp.int32)]).reshape(NS, K, CH)
    zeros = jnp.zeros((CH, HALF), jnp.float32)

    table = _make_table(hv, W2, b2)
    eu = _sc_scatter(table, src_pad, dst_pad, zeros)
    g2 = gamma.reshape(NC, HALF)
    be2 = beta.reshape(NC, HALF)
    return _batchnorm(eu, hv, g2, be2)
